# baseline (device time: 28799 ns/iter reference)
import functools

import jax
import jax.numpy as jnp
from jax import lax
from jax.experimental import pallas as pl
from jax.experimental.pallas import tpu as pltpu

N_DEV = 32
LOG2_N = 5
B, SQ, D_MODEL = 2, 128, 512
HQ, DH = 4, 64
D_QK = HQ * DH
LW = D_QK + 32 * HQ
SKV_LOC = 128
BLK = 64


def kernel(x, Wq, K_ext, V_ext, Wo):
    kT = jnp.transpose(K_ext, (0, 2, 1, 3))
    vT = jnp.transpose(V_ext, (0, 2, 1, 3))

    def body(x_ref, wq_ref, k_ref, v_ref, wo_ref, out_ref,
             q_ref, acc_ref, send_ref, recv_ref, send_sems, recv_sems,
             exit_sem):
        my = lax.axis_index("i")

        live = jnp.where((my % 2) == 0, 1.0, 0.0).astype(jnp.float32)
        ones = jnp.ones((BLK, 32), jnp.float32)

        def partial_attn(b):
            q_ref[b] = jnp.dot(
                x_ref[b].astype(jnp.bfloat16),
                wq_ref[...].astype(jnp.bfloat16),
                preferred_element_type=jnp.float32,
            )
            for h in range(HQ):
                for g in range(2):
                    rows = slice(g * BLK, (g + 1) * BLK)
                    qg = q_ref[b, rows, h * DH:(h + 1) * DH]
                    s = lax.dot_general(
                        qg, k_ref[b, h, rows],
                        (((1,), (1,)), ((), ())),
                        preferred_element_type=jnp.float32,
                    ) * 0.125
                    w = jnp.exp(s) * live
                    o = jnp.dot(w, v_ref[b, h, rows],
                                preferred_element_type=jnp.float32)
                    lsum = jnp.dot(w, ones,
                                   preferred_element_type=jnp.float32)
                    acc_ref[b, rows, h * DH:(h + 1) * DH] = o
                    acc_ref[b, rows, D_QK + 32 * h:D_QK + 32 * h + 32] = lsum

        HSQ = SQ // 2

        def make_rdma(kstep, b, j):
            p = my ^ (1 << kstep)
            return pltpu.make_async_remote_copy(
                src_ref=send_ref.at[b, pl.ds(j * HSQ, HSQ)],
                dst_ref=recv_ref.at[kstep, b, pl.ds(j * HSQ, HSQ)],
                send_sem=send_sems.at[kstep, b, j],
                recv_sem=recv_sems.at[kstep, b, j],
                device_id=(p,),
                device_id_type=pl.DeviceIdType.MESH,
            )

        def cast_and_send(kstep, b, j):
            send_ref[b, j * HSQ:(j + 1) * HSQ] = (
                acc_ref[b, j * HSQ:(j + 1) * HSQ].astype(jnp.bfloat16))
            make_rdma(kstep, b, j).start()

        barrier = pltpu.get_barrier_semaphore()
        for kstep in range(LOG2_N):
            p = my ^ (1 << kstep)
            pl.semaphore_signal(barrier, inc=1, device_id=(p,),
                                device_id_type=pl.DeviceIdType.MESH)

        partial_attn(0)
        pl.semaphore_wait(barrier, LOG2_N)

        cast_and_send(0, 0, 0)
        cast_and_send(0, 0, 1)
        partial_attn(1)
        cast_and_send(0, 1, 0)
        cast_and_send(0, 1, 1)

        def finalize(b):
            for h in range(HQ):
                o = acc_ref[b, :, h * DH:(h + 1) * DH]
                denom = acc_ref[b, :, D_QK + 32 * h:D_QK + 32 * h + 1]
                q_ref[b, :, h * DH:(h + 1) * DH] = o / denom
            out_ref[b] = jnp.dot(
                q_ref[b].astype(jnp.bfloat16),
                wo_ref[...].astype(jnp.bfloat16),
                preferred_element_type=jnp.float32,
            )

        for kstep in range(LOG2_N):
            for b in range(B):
                for j in range(2):
                    make_rdma(kstep, b, j).wait()
                    sl = slice(j * HSQ, (j + 1) * HSQ)
                    acc_ref[b, sl] = (
                        acc_ref[b, sl]
                        + recv_ref[kstep, b, sl].astype(jnp.float32))
                    if kstep + 1 < LOG2_N:
                        cast_and_send(kstep + 1, b, j)
                    elif j == 1:
                        if b == 1:
                            for ks in range(LOG2_N):
                                p = my ^ (1 << ks)
                                pl.semaphore_signal(
                                    exit_sem, inc=1, device_id=(p,),
                                    device_id_type=pl.DeviceIdType.MESH)
                        finalize(b)

        pl.semaphore_wait(exit_sem, LOG2_N)

    return pl.pallas_call(
        body,
        out_shape=jax.ShapeDtypeStruct((B, SQ, D_MODEL), jnp.float32),
        in_specs=[pl.BlockSpec(memory_space=pltpu.VMEM)] * 5,
        out_specs=pl.BlockSpec(memory_space=pltpu.VMEM),
        scratch_shapes=[
            pltpu.VMEM((B, SQ, D_QK), jnp.float32),
            pltpu.VMEM((B, SQ, LW), jnp.float32),
            pltpu.VMEM((B, SQ, LW), jnp.bfloat16),
            pltpu.VMEM((LOG2_N, B, SQ, LW), jnp.bfloat16),
            pltpu.SemaphoreType.DMA((LOG2_N, B, 2)),
            pltpu.SemaphoreType.DMA((LOG2_N, B, 2)),
            pltpu.SemaphoreType.REGULAR,
        ],
        compiler_params=pltpu.CompilerParams(collective_id=0),
    )(x, Wq, kT, vT, Wo)


# device time: 28744 ns/iter; 1.0019x vs baseline; 1.0019x over previous
import functools

import jax
import jax.numpy as jnp
from jax import lax
from jax.experimental import pallas as pl
from jax.experimental.pallas import tpu as pltpu

N_DEV = 32
LOG2_N = 5
MASKS = (16, 3, 4, 8, 1)
B, SQ, D_MODEL = 2, 128, 512
HQ, DH = 4, 64
D_QK = HQ * DH
LW = D_QK + 32 * HQ
SKV_LOC = 128
BLK = 64


def kernel(x, Wq, K_ext, V_ext, Wo):
    kT = jnp.transpose(K_ext, (0, 2, 1, 3))
    vT = jnp.transpose(V_ext, (0, 2, 1, 3))

    def body(x_ref, wq_ref, k_ref, v_ref, wo_ref, out_ref,
             q_ref, acc_ref, send_ref, recv_ref, send_sems, recv_sems,
             exit_sem):
        my = lax.axis_index("i")

        live = jnp.where((my % 2) == 0, 1.0, 0.0).astype(jnp.float32)
        ones = jnp.ones((BLK, 32), jnp.float32)

        def partial_attn(b):
            q_ref[b] = jnp.dot(
                x_ref[b].astype(jnp.bfloat16),
                wq_ref[...].astype(jnp.bfloat16),
                preferred_element_type=jnp.float32,
            )
            for h in range(HQ):
                for g in range(2):
                    rows = slice(g * BLK, (g + 1) * BLK)
                    qg = q_ref[b, rows, h * DH:(h + 1) * DH]
                    s = lax.dot_general(
                        qg, k_ref[b, h, rows],
                        (((1,), (1,)), ((), ())),
                        preferred_element_type=jnp.float32,
                    ) * 0.125
                    w = jnp.exp(s) * live
                    o = jnp.dot(w, v_ref[b, h, rows],
                                preferred_element_type=jnp.float32)
                    lsum = jnp.dot(w, ones,
                                   preferred_element_type=jnp.float32)
                    acc_ref[b, rows, h * DH:(h + 1) * DH] = o
                    acc_ref[b, rows, D_QK + 32 * h:D_QK + 32 * h + 32] = lsum

        HSQ = SQ // 2

        def make_rdma(kstep, b, j):
            p = my ^ MASKS[kstep]
            return pltpu.make_async_remote_copy(
                src_ref=send_ref.at[b, pl.ds(j * HSQ, HSQ)],
                dst_ref=recv_ref.at[kstep, b, pl.ds(j * HSQ, HSQ)],
                send_sem=send_sems.at[kstep, b, j],
                recv_sem=recv_sems.at[kstep, b, j],
                device_id=(p,),
                device_id_type=pl.DeviceIdType.MESH,
            )

        def cast_and_send(kstep, b, j):
            send_ref[b, j * HSQ:(j + 1) * HSQ] = (
                acc_ref[b, j * HSQ:(j + 1) * HSQ].astype(jnp.bfloat16))
            make_rdma(kstep, b, j).start()

        barrier = pltpu.get_barrier_semaphore()
        for m in MASKS:
            pl.semaphore_signal(barrier, inc=1, device_id=(my ^ m,),
                                device_id_type=pl.DeviceIdType.MESH)

        partial_attn(0)
        pl.semaphore_wait(barrier, LOG2_N)

        cast_and_send(0, 0, 0)
        cast_and_send(0, 0, 1)
        partial_attn(1)
        cast_and_send(0, 1, 0)
        cast_and_send(0, 1, 1)

        def finalize(b):
            for h in range(HQ):
                o = acc_ref[b, :, h * DH:(h + 1) * DH]
                denom = acc_ref[b, :, D_QK + 32 * h:D_QK + 32 * h + 1]
                q_ref[b, :, h * DH:(h + 1) * DH] = o / denom
            out_ref[b] = jnp.dot(
                q_ref[b].astype(jnp.bfloat16),
                wo_ref[...].astype(jnp.bfloat16),
                preferred_element_type=jnp.float32,
            )

        for kstep in range(LOG2_N):
            for b in range(B):
                for j in range(2):
                    make_rdma(kstep, b, j).wait()
                    sl = slice(j * HSQ, (j + 1) * HSQ)
                    acc_ref[b, sl] = (
                        acc_ref[b, sl]
                        + recv_ref[kstep, b, sl].astype(jnp.float32))
                    if kstep + 1 < LOG2_N:
                        cast_and_send(kstep + 1, b, j)
                    elif j == 1:
                        if b == 1:
                            for m in MASKS:
                                pl.semaphore_signal(
                                    exit_sem, inc=1, device_id=(my ^ m,),
                                    device_id_type=pl.DeviceIdType.MESH)
                        finalize(b)

        pl.semaphore_wait(exit_sem, LOG2_N)

    return pl.pallas_call(
        body,
        out_shape=jax.ShapeDtypeStruct((B, SQ, D_MODEL), jnp.float32),
        in_specs=[pl.BlockSpec(memory_space=pltpu.VMEM)] * 5,
        out_specs=pl.BlockSpec(memory_space=pltpu.VMEM),
        scratch_shapes=[
            pltpu.VMEM((B, SQ, D_QK), jnp.float32),
            pltpu.VMEM((B, SQ, LW), jnp.float32),
            pltpu.VMEM((B, SQ, LW), jnp.bfloat16),
            pltpu.VMEM((LOG2_N, B, SQ, LW), jnp.bfloat16),
            pltpu.SemaphoreType.DMA((LOG2_N, B, 2)),
            pltpu.SemaphoreType.DMA((LOG2_N, B, 2)),
            pltpu.SemaphoreType.REGULAR,
        ],
        compiler_params=pltpu.CompilerParams(collective_id=0),
    )(x, Wq, kT, vT, Wo)


# device time: 27397 ns/iter; 1.0512x vs baseline; 1.0492x over previous
import functools

import jax
import jax.numpy as jnp
from jax import lax
from jax.experimental import pallas as pl
from jax.experimental.pallas import tpu as pltpu

N_DEV = 32
LOG2_N = 5
MASKS = (16, 3, 4, 8, 1)
B, SQ, D_MODEL = 2, 128, 512
HQ, DH = 4, 64
D_QK = HQ * DH
LW = D_QK + 32 * HQ
SKV_LOC = 128
BLK = 64


def kernel(x, Wq, K_ext, V_ext, Wo):
    kT = jnp.transpose(K_ext, (0, 2, 1, 3))
    vT = jnp.transpose(V_ext, (0, 2, 1, 3))

    def body(x_ref, wq_ref, k_ref, v_ref, wo_ref, out_ref,
             q_ref, acc_ref, send_ref, recv_ref, send_sems, recv_sems,
             exit_sem):
        my = lax.axis_index("i")

        live = jnp.where((my % 2) == 0, 1.0, 0.0).astype(jnp.float32)
        ones = jnp.ones((BLK, 32), jnp.float32)

        def partial_attn(b):
            q_ref[b] = jnp.dot(
                x_ref[b].astype(jnp.bfloat16),
                wq_ref[...].astype(jnp.bfloat16),
                preferred_element_type=jnp.float32,
            )
            for h in range(HQ):
                for g in range(2):
                    rows = slice(g * BLK, (g + 1) * BLK)
                    qg = q_ref[b, rows, h * DH:(h + 1) * DH]
                    s = lax.dot_general(
                        qg, k_ref[b, h, rows],
                        (((1,), (1,)), ((), ())),
                        preferred_element_type=jnp.float32,
                    ) * 0.125
                    w = jnp.exp(s) * live
                    o = jnp.dot(w, v_ref[b, h, rows],
                                preferred_element_type=jnp.float32)
                    lsum = jnp.dot(w, ones,
                                   preferred_element_type=jnp.float32)
                    acc_ref[b, rows, h * DH:(h + 1) * DH] = o
                    acc_ref[b, rows, D_QK + 32 * h:D_QK + 32 * h + 32] = lsum

        HSQ = SQ // 2

        def make_rdma(kstep, b, j):
            p = my ^ MASKS[kstep]
            return pltpu.make_async_remote_copy(
                src_ref=send_ref.at[kstep, b, pl.ds(j * HSQ, HSQ)],
                dst_ref=recv_ref.at[kstep, b, pl.ds(j * HSQ, HSQ)],
                send_sem=send_sems.at[kstep, b, j],
                recv_sem=recv_sems.at[kstep, b, j],
                device_id=(p,),
                device_id_type=pl.DeviceIdType.MESH,
            )

        def cast_and_send(kstep, b, j):
            send_ref[kstep, b, j * HSQ:(j + 1) * HSQ] = (
                acc_ref[b, j * HSQ:(j + 1) * HSQ].astype(jnp.bfloat16))
            make_rdma(kstep, b, j).start()

        barrier = pltpu.get_barrier_semaphore()
        for m in MASKS:
            pl.semaphore_signal(barrier, inc=1, device_id=(my ^ m,),
                                device_id_type=pl.DeviceIdType.MESH)

        partial_attn(0)
        pl.semaphore_wait(barrier, LOG2_N)

        cast_and_send(0, 0, 0)
        cast_and_send(0, 0, 1)
        partial_attn(1)
        cast_and_send(0, 1, 0)
        cast_and_send(0, 1, 1)

        def finalize(b):
            for h in range(HQ):
                o = acc_ref[b, :, h * DH:(h + 1) * DH]
                denom = acc_ref[b, :, D_QK + 32 * h:D_QK + 32 * h + 1]
                q_ref[b, :, h * DH:(h + 1) * DH] = o / denom
            out_ref[b] = jnp.dot(
                q_ref[b].astype(jnp.bfloat16),
                wo_ref[...].astype(jnp.bfloat16),
                preferred_element_type=jnp.float32,
            )

        for kstep in range(LOG2_N):
            for b in range(B):
                for j in range(2):
                    make_rdma(kstep, b, j).wait_recv()
                    sl = slice(j * HSQ, (j + 1) * HSQ)
                    acc_ref[b, sl] = (
                        acc_ref[b, sl]
                        + recv_ref[kstep, b, sl].astype(jnp.float32))
                    if kstep + 1 < LOG2_N:
                        cast_and_send(kstep + 1, b, j)
                    elif j == 1:
                        if b == 1:
                            for m in MASKS:
                                pl.semaphore_signal(
                                    exit_sem, inc=1, device_id=(my ^ m,),
                                    device_id_type=pl.DeviceIdType.MESH)
                        finalize(b)

        for kstep in range(LOG2_N):
            for b in range(B):
                for j in range(2):
                    make_rdma(kstep, b, j).wait_send()

        pl.semaphore_wait(exit_sem, LOG2_N)

    return pl.pallas_call(
        body,
        out_shape=jax.ShapeDtypeStruct((B, SQ, D_MODEL), jnp.float32),
        in_specs=[pl.BlockSpec(memory_space=pltpu.VMEM)] * 5,
        out_specs=pl.BlockSpec(memory_space=pltpu.VMEM),
        scratch_shapes=[
            pltpu.VMEM((B, SQ, D_QK), jnp.float32),
            pltpu.VMEM((B, SQ, LW), jnp.float32),
            pltpu.VMEM((LOG2_N, B, SQ, LW), jnp.bfloat16),
            pltpu.VMEM((LOG2_N, B, SQ, LW), jnp.bfloat16),
            pltpu.SemaphoreType.DMA((LOG2_N, B, 2)),
            pltpu.SemaphoreType.DMA((LOG2_N, B, 2)),
            pltpu.SemaphoreType.REGULAR,
        ],
        compiler_params=pltpu.CompilerParams(collective_id=0),
    )(x, Wq, kT, vT, Wo)
